# baseline (speedup 1.0000x reference)
import functools

import jax
import jax.numpy as jnp
from jax import lax
from jax.experimental import pallas as pl
from jax.experimental.pallas import tpu as pltpu
from jax.experimental.pallas import tpu_sc as plsc

_NC = 2
_NS = 16
_NW = _NC * _NS
_CHUNK = 2048
_ROWS = _CHUNK // 128


def _mesh():
    return plsc.VectorSubcoreMesh(core_axis_name="c", subcore_axis_name="s",
                                  num_cores=_NC, num_subcores=_NS)


@functools.partial(jax.jit, static_argnames=("with_deg", "NP"))
def _edge_pass(table, ei, *, with_deg, NP):
    TN = table.shape[0]
    E = ei.shape[1]
    F, rem = divmod(E, _CHUNK)
    assert rem % 128 == 0, "edge count must be a multiple of 128"
    n_pipe = F // _NW
    n_left = F - n_pipe * _NW
    words = NP // _NS

    out_type = [jax.ShapeDtypeStruct((_NC, NP), jnp.float32)]
    if with_deg:
        out_type.append(jax.ShapeDtypeStruct((_NC, NP), jnp.float32))

    NB = 2
    scratch = (
        [pltpu.VMEM((TN,), jnp.float32)]
        + [pltpu.VMEM((_CHUNK,), jnp.int32) for _ in range(NB)]
        + [pltpu.VMEM((_CHUNK,), jnp.int32) for _ in range(NB)]
        + [pltpu.VMEM((_CHUNK,), jnp.float32) for _ in range(NB)]
        + [pltpu.VMEM((_CHUNK,), jnp.float32),
           pltpu.VMEM_SHARED((NP,), jnp.float32)]
        + ([pltpu.VMEM_SHARED((NP,), jnp.float32)] if with_deg else [])
        + [pltpu.SemaphoreType.DMA for _ in range(2 * NB)]
    )

    def body(table_hbm, ei_hbm, *rest):
        rest = list(rest)
        out_s_hbm = rest.pop(0)
        out_d_hbm = rest.pop(0) if with_deg else None
        tbl = rest.pop(0)
        srcb = [rest.pop(0) for _ in range(NB)]
        dstb = [rest.pop(0) for _ in range(NB)]
        valb = [rest.pop(0) for _ in range(NB)]
        util = rest.pop(0)
        acc_s = rest.pop(0)
        acc_d = rest.pop(0) if with_deg else None
        lsem = [rest.pop(0) for _ in range(NB)]
        ssem = [rest.pop(0) for _ in range(NB)]
        assert not rest

        cid = lax.axis_index("c")
        sid = lax.axis_index("s")
        dyn1 = jnp.minimum(cid, 0) + 1

        def fillz(i, _):
            util[pl.ds(i * 16, 16)] = jnp.zeros((16,), jnp.float32)
            return 0
        lax.fori_loop(0, _CHUNK // 16, fillz, 0)

        base = sid * words
        n_full_z, rem_z = divmod(words, _CHUNK)
        for k in range(n_full_z):
            pltpu.sync_copy(util, acc_s.at[pl.ds(base + k * _CHUNK, _CHUNK)])
            if with_deg:
                pltpu.sync_copy(util,
                                acc_d.at[pl.ds(base + k * _CHUNK, _CHUNK)])
        if rem_z:
            pltpu.sync_copy(util.at[pl.ds(0, rem_z)],
                            acc_s.at[pl.ds(base + n_full_z * _CHUNK, rem_z)])
            if with_deg:
                pltpu.sync_copy(
                    util.at[pl.ds(0, rem_z)],
                    acc_d.at[pl.ds(base + n_full_z * _CHUNK, rem_z)])

        pltpu.sync_copy(table_hbm, tbl)

        def fillo(i, _):
            util[pl.ds(i * 16, 16)] = jnp.full((16,), 1.0, jnp.float32)
            return 0
        lax.fori_loop(0, _CHUNK // 16, fillo, 0)
        plsc.subcore_barrier()

        w = cid * _NS + sid

        def gather_chunk(sv, vv, n_edges):
            def g(i, sv=sv, vv=vv):
                vv[pl.ds(i, 16)] = plsc.load_gather(tbl, [sv[pl.ds(i, 16)]])
            plsc.parallel_loop(0, n_edges, step=16, unroll=8)(g)

        def fire_scatters(vv, dv, sem, n_edges):
            idx = dv.at[pl.ds(0, n_edges)]
            cps = [pltpu.async_copy(
                vv.at[pl.ds(0, n_edges)], acc_s.at[idx], sem, add=True)]
            if with_deg:
                cps.append(pltpu.async_copy(
                    util.at[pl.ds(0, n_edges)], acc_d.at[idx],
                    sem, add=True))
            return cps

        load_descs = {}
        scat_descs = {}

        def start_loads(ci):
            chunk_id = w * n_pipe + ci
            b = ci % NB
            d1 = pltpu.async_copy(
                ei_hbm.at[0, pl.ds(chunk_id * _CHUNK, _CHUNK)], srcb[b],
                lsem[b])
            d2 = pltpu.async_copy(
                ei_hbm.at[dyn1, pl.ds(chunk_id * _CHUNK, _CHUNK)], dstb[b],
                lsem[b])
            load_descs[ci] = [d1, d2]

        start_loads(0)
        for ci in range(n_pipe):
            b = ci % NB
            for d in load_descs.pop(ci):
                d.wait()
            if ci + 1 < n_pipe:
                if ci - 1 >= 0:
                    for d in scat_descs.pop(ci - 1):
                        d.wait()
                start_loads(ci + 1)
            gather_chunk(srcb[b], valb[b], _CHUNK)
            scat_descs[ci] = fire_scatters(valb[b], dstb[b], ssem[b], _CHUNK)
        for ci in sorted(scat_descs):
            for d in scat_descs[ci]:
                d.wait()

        left_id = n_pipe * _NW + w

        def do_tail(n_edges):
            def tail():
                ebase = left_id * _CHUNK
                pltpu.sync_copy(ei_hbm.at[0, pl.ds(ebase, n_edges)],
                                srcb[0].at[pl.ds(0, n_edges)])
                pltpu.sync_copy(ei_hbm.at[dyn1, pl.ds(ebase, n_edges)],
                                dstb[0].at[pl.ds(0, n_edges)])
                gather_chunk(srcb[0], valb[0], n_edges)
                for d in fire_scatters(valb[0], dstb[0], ssem[0], n_edges):
                    d.wait()
            return tail

        if n_left:
            pl.when(left_id < F)(do_tail(_CHUNK))
        if rem:
            pl.when(left_id == F)(do_tail(rem))

        plsc.subcore_barrier()
        pltpu.sync_copy(acc_s.at[pl.ds(base, words)],
                        out_s_hbm.at[cid, pl.ds(base, words)])
        if with_deg:
            pltpu.sync_copy(acc_d.at[pl.ds(base, words)],
                            out_d_hbm.at[cid, pl.ds(base, words)])

    k = pl.kernel(body, out_type=out_type, mesh=_mesh(),
                  scratch_types=scratch,
                  compiler_params=pltpu.CompilerParams(
                      needs_layout_passes=False))
    return k(table, ei)


def _combine(S2, D2, Wl1, Wr1, b1, Wl2, Wr2, b2):
    NP = S2.shape[1]
    H2 = Wl2.shape[1]

    def body(s2, d2, wl1, wr1, b1r, wl2, wr2, b2r,
             mean1_o, gamma_o, dinv_o, u_o):
        s = s2[0, :] + s2[1, :]
        deg = d2[0, :] + d2[1, :]
        dm = jnp.maximum(deg, 1.0)
        mean1_o[:] = s / dm
        gamma_o[:] = deg / dm
        dinv_o[:] = 1.0 / dm
        u1 = jnp.dot(wl1[:], wl2[:])
        u24 = jnp.dot(wr1[:], wl2[:]) + jnp.dot(wl1[:], wr2[:])
        u3 = jnp.dot(b1r[:][None, :], wl2[:])
        u5 = jnp.dot(wr1[:], wr2[:])
        c0 = jnp.dot(b1r[:][None, :], wr2[:]) + b2r[:][None, :]
        z = jnp.zeros((3, H2), jnp.float32)
        u_o[:, :] = jnp.concatenate([u1, u24, u3, u5, c0, z], axis=0)

    return pl.pallas_call(
        body,
        out_shape=[
            jax.ShapeDtypeStruct((NP,), jnp.float32),
            jax.ShapeDtypeStruct((NP,), jnp.float32),
            jax.ShapeDtypeStruct((NP,), jnp.float32),
            jax.ShapeDtypeStruct((8, H2), jnp.float32),
        ],
    )(S2, D2, Wl1, Wr1, b1, Wl2, Wr2, b2)


@functools.partial(jax.jit, static_argnames=("kpad",))
def _pool(A0, A1, mean1, gamma, dinv, xv, cl, U, *, kpad):
    NP = mean1.shape[0]
    per = NP // _NW

    scratch = [
        pltpu.VMEM((per,), jnp.float32),
        pltpu.VMEM((per,), jnp.float32),
        pltpu.VMEM((per,), jnp.float32),
        pltpu.VMEM((per,), jnp.float32),
        pltpu.VMEM((per,), jnp.float32),
        pltpu.VMEM((per,), jnp.float32),
        pltpu.VMEM((per,), jnp.int32),
        pltpu.VMEM((8, 32), jnp.float32),
        pltpu.VMEM((kpad, 32), jnp.float32),
    ]

    def body(a0_hbm, a1_hbm, m1_hbm, ga_hbm, di_hbm, x_hbm, cl_hbm, u_hbm,
             out_hbm, a0v, a1v, m1v, gav, div, xvv, clv, uv, pooled):
        cid = lax.axis_index("c")
        sid = lax.axis_index("s")
        w = cid * _NS + sid
        base = w * per
        pltpu.sync_copy(a0_hbm.at[pl.ds(base, per)], a0v)
        pltpu.sync_copy(a1_hbm.at[pl.ds(base, per)], a1v)
        pltpu.sync_copy(m1_hbm.at[pl.ds(base, per)], m1v)
        pltpu.sync_copy(ga_hbm.at[pl.ds(base, per)], gav)
        pltpu.sync_copy(di_hbm.at[pl.ds(base, per)], div)
        pltpu.sync_copy(x_hbm.at[pl.ds(base, per)], xvv)
        pltpu.sync_copy(cl_hbm.at[pl.ds(base, per)], clv)
        pltpu.sync_copy(u_hbm, uv)

        ninf = jnp.full((16,), -jnp.inf, jnp.float32)

        def pinit(k, _):
            pooled[k, pl.ds(0, 16)] = ninf
            pooled[k, pl.ds(16, 16)] = ninf
            return 0
        lax.fori_loop(0, kpad, pinit, 0)

        u1a = uv[0, pl.ds(0, 16)]
        u1b = uv[0, pl.ds(16, 16)]
        u24a = uv[1, pl.ds(0, 16)]
        u24b = uv[1, pl.ds(16, 16)]
        u3a = uv[2, pl.ds(0, 16)]
        u3b = uv[2, pl.ds(16, 16)]
        u5a = uv[3, pl.ds(0, 16)]
        u5b = uv[3, pl.ds(16, 16)]

        def nb(i, carry):
            b16 = i * 16
            alv = ((a0v[pl.ds(b16, 16)] + a1v[pl.ds(b16, 16)])
                   * div[pl.ds(b16, 16)])
            bev = m1v[pl.ds(b16, 16)]
            gav16 = gav[pl.ds(b16, 16)]
            xxv = xvv[pl.ds(b16, 16)]
            clv16 = clv[pl.ds(b16, 16)]
            c_first = clv16[0]
            c_last = clv16[15]

            def fast(carry):
                cprev, m0, m1 = carry

                def do_flush(cp=cprev, a=m0, b=m1):
                    pooled[cp, pl.ds(0, 16)] = a
                    pooled[cp, pl.ds(16, 16)] = b
                    return ninf, ninf

                def no_flush(a=m0, b=m1):
                    return a, b
                m0, m1 = lax.cond(c_first != cprev, do_flush, no_flush)
                for j in range(16):
                    al = alv[j]
                    be = bev[j]
                    ga = gav16[j]
                    xx = xxv[j]
                    h0 = (al * u1a + be * u24a) + (ga * u3a + xx * u5a)
                    h1 = (al * u1b + be * u24b) + (ga * u3b + xx * u5b)
                    m0 = jnp.maximum(m0, h0)
                    m1 = jnp.maximum(m1, h1)
                return c_first, m0, m1

            def slow(carry):
                cprev, m0, m1 = carry
                for j in range(16):
                    al = alv[j]
                    be = bev[j]
                    ga = gav16[j]
                    xx = xxv[j]
                    c = clv16[j]
                    h0 = (al * u1a + be * u24a) + (ga * u3a + xx * u5a)
                    h1 = (al * u1b + be * u24b) + (ga * u3b + xx * u5b)
                    flush = c != cprev

                    def do_flush(cp=cprev, a=m0, b=m1):
                        pooled[cp, pl.ds(0, 16)] = a
                        pooled[cp, pl.ds(16, 16)] = b
                        return ninf, ninf

                    def no_flush(a=m0, b=m1):
                        return a, b
                    m0, m1 = lax.cond(flush, do_flush, no_flush)
                    m0 = jnp.maximum(m0, h0)
                    m1 = jnp.maximum(m1, h1)
                    cprev = c
                return cprev, m0, m1

            return lax.cond(c_first == c_last, fast, slow, carry)
        cprev, m0, m1 = lax.fori_loop(
            0, per // 16, nb,
            (jnp.int32(kpad - 1), ninf, ninf))
        pooled[cprev, pl.ds(0, 16)] = m0
        pooled[cprev, pl.ds(16, 16)] = m1

        pltpu.sync_copy(pooled, out_hbm.at[w])

    k = pl.kernel(
        body,
        out_type=jax.ShapeDtypeStruct((_NW, kpad, 32), jnp.float32),
        mesh=_mesh(),
        scratch_types=scratch,
        compiler_params=pltpu.CompilerParams(needs_layout_passes=False),
    )
    return k(A0, A1, mean1, gamma, dinv, xv, cl, U)


def _head(p32, nclusters, We3, be, Wp1, bp1, Wp2, bp2, U):
    def body(p, we3, ber, wp1, bp1r, wp2, bp2r, u, out):
        pm = jnp.max(p[...], axis=0)
        pm = pm[:nclusters, :]
        pm = jnp.where(jnp.isfinite(pm), pm + u[4, :][None, :], 0.0)
        t = pm[:, :, None] * we3[...]
        emb = jnp.sum(jnp.sum(t, axis=0), axis=0, keepdims=True)
        emb = jnp.maximum(emb + ber[:][None, :], 0.0)
        p1 = jnp.maximum(jnp.dot(emb, wp1[...]) + bp1r[:][None, :], 0.0)
        out[:, :] = jax.nn.sigmoid(jnp.dot(p1, wp2[...]) + bp2r[:][None, :])

    return pl.pallas_call(
        body,
        out_shape=jax.ShapeDtypeStruct((1, 1), jnp.float32),
    )(p32, We3, be, Wp1, bp1, Wp2, bp2, U)


def kernel(x, edge_index, clusters, Wl1, Wr1, b1, Wl2, Wr2, b2,
           We, be, Wp1, bp1, Wp2, bp2):
    N = x.shape[0]
    H2 = Wl2.shape[1]
    K = We.shape[0] // H2
    R = We.shape[1]

    NP = ((N + 2047) // 2048) * 2048
    kpad = ((K + 1 + 15) // 16) * 16

    xv = x[:, 0]
    xp = jnp.concatenate([xv, jnp.zeros((NP - N,), jnp.float32)])
    clp = jnp.concatenate(
        [clusters, jnp.full((NP - N,), K, jnp.int32)])

    S2, D2 = _edge_pass(xv, edge_index, with_deg=True, NP=NP)
    mean1, gamma, dinv, U = _combine(S2, D2, Wl1, Wr1, b1, Wl2, Wr2, b2)
    (A2,) = _edge_pass(mean1, edge_index, with_deg=False, NP=NP)
    p32 = _pool(A2[0], A2[1], mean1, gamma, dinv, xp, clp, U, kpad=kpad)
    We3 = We.reshape(K, H2, R)
    return _head(p32, K, We3, be, Wp1, bp1, Wp2, bp2, U)


# speedup vs baseline: 1.0436x; 1.0436x over previous
import functools

import jax
import jax.numpy as jnp
from jax import lax
from jax.experimental import pallas as pl
from jax.experimental.pallas import tpu as pltpu
from jax.experimental.pallas import tpu_sc as plsc

_NC = 2
_NS = 16
_NW = _NC * _NS
_CHUNK = 2048
_ROWS = _CHUNK // 128


def _mesh():
    return plsc.VectorSubcoreMesh(core_axis_name="c", subcore_axis_name="s",
                                  num_cores=_NC, num_subcores=_NS)


@functools.partial(jax.jit, static_argnames=("with_deg", "NP"))
def _edge_pass(table, ei, *, with_deg, NP):
    TN = table.shape[0]
    E = ei.shape[1]
    F, rem = divmod(E, _CHUNK)
    assert rem % 128 == 0, "edge count must be a multiple of 128"
    n_pipe = F // _NW
    n_left = F - n_pipe * _NW
    words = NP // _NS

    out_type = [jax.ShapeDtypeStruct((_NC, NP), jnp.float32)]
    if with_deg:
        out_type.append(jax.ShapeDtypeStruct((_NC, NP), jnp.float32))

    NB = 2
    scratch = (
        [pltpu.VMEM((TN,), jnp.float32)]
        + [pltpu.VMEM((_CHUNK,), jnp.int32) for _ in range(NB)]
        + [pltpu.VMEM((_CHUNK,), jnp.int32) for _ in range(NB)]
        + [pltpu.VMEM((_CHUNK,), jnp.float32) for _ in range(NB)]
        + [pltpu.VMEM((_CHUNK,), jnp.float32),
           pltpu.VMEM_SHARED((NP,), jnp.float32)]
        + ([pltpu.VMEM_SHARED((NP,), jnp.float32)] if with_deg else [])
        + [pltpu.SemaphoreType.DMA for _ in range(2 * NB)]
    )

    def body(table_hbm, ei_hbm, *rest):
        rest = list(rest)
        out_s_hbm = rest.pop(0)
        out_d_hbm = rest.pop(0) if with_deg else None
        tbl = rest.pop(0)
        srcb = [rest.pop(0) for _ in range(NB)]
        dstb = [rest.pop(0) for _ in range(NB)]
        valb = [rest.pop(0) for _ in range(NB)]
        util = rest.pop(0)
        acc_s = rest.pop(0)
        acc_d = rest.pop(0) if with_deg else None
        lsem = [rest.pop(0) for _ in range(NB)]
        ssem = [rest.pop(0) for _ in range(NB)]
        assert not rest

        cid = lax.axis_index("c")
        sid = lax.axis_index("s")
        dyn1 = jnp.minimum(cid, 0) + 1

        def fillz(i, _):
            util[pl.ds(i * 16, 16)] = jnp.zeros((16,), jnp.float32)
            return 0
        lax.fori_loop(0, _CHUNK // 16, fillz, 0)

        base = sid * words
        n_full_z, rem_z = divmod(words, _CHUNK)
        for k in range(n_full_z):
            pltpu.sync_copy(util, acc_s.at[pl.ds(base + k * _CHUNK, _CHUNK)])
            if with_deg:
                pltpu.sync_copy(util,
                                acc_d.at[pl.ds(base + k * _CHUNK, _CHUNK)])
        if rem_z:
            pltpu.sync_copy(util.at[pl.ds(0, rem_z)],
                            acc_s.at[pl.ds(base + n_full_z * _CHUNK, rem_z)])
            if with_deg:
                pltpu.sync_copy(
                    util.at[pl.ds(0, rem_z)],
                    acc_d.at[pl.ds(base + n_full_z * _CHUNK, rem_z)])

        pltpu.sync_copy(table_hbm, tbl)

        def fillo(i, _):
            util[pl.ds(i * 16, 16)] = jnp.full((16,), 1.0, jnp.float32)
            return 0
        lax.fori_loop(0, _CHUNK // 16, fillo, 0)
        plsc.subcore_barrier()

        w = cid * _NS + sid

        def gather_chunk(sv, vv, n_edges):
            def g(i, sv=sv, vv=vv):
                vv[pl.ds(i, 16)] = plsc.load_gather(tbl, [sv[pl.ds(i, 16)]])
            plsc.parallel_loop(0, n_edges, step=16, unroll=8)(g)

        def fire_scatters(vv, dv, sem, n_edges):
            idx = dv.at[pl.ds(0, n_edges)]
            cps = [pltpu.async_copy(
                vv.at[pl.ds(0, n_edges)], acc_s.at[idx], sem, add=True)]
            if with_deg:
                cps.append(pltpu.async_copy(
                    util.at[pl.ds(0, n_edges)], acc_d.at[idx],
                    sem, add=True))
            return cps

        load_descs = {}
        scat_descs = {}

        def start_loads(ci):
            chunk_id = w * n_pipe + ci
            b = ci % NB
            d1 = pltpu.async_copy(
                ei_hbm.at[0, pl.ds(chunk_id * _CHUNK, _CHUNK)], srcb[b],
                lsem[b])
            d2 = pltpu.async_copy(
                ei_hbm.at[dyn1, pl.ds(chunk_id * _CHUNK, _CHUNK)], dstb[b],
                lsem[b])
            load_descs[ci] = [d1, d2]

        start_loads(0)
        for ci in range(n_pipe):
            b = ci % NB
            for d in load_descs.pop(ci):
                d.wait()
            if ci + 1 < n_pipe:
                if ci - 1 >= 0:
                    for d in scat_descs.pop(ci - 1):
                        d.wait()
                start_loads(ci + 1)
            gather_chunk(srcb[b], valb[b], _CHUNK)
            scat_descs[ci] = fire_scatters(valb[b], dstb[b], ssem[b], _CHUNK)
        for ci in sorted(scat_descs):
            for d in scat_descs[ci]:
                d.wait()

        left_id = n_pipe * _NW + w

        def do_tail(n_edges):
            def tail():
                ebase = left_id * _CHUNK
                pltpu.sync_copy(ei_hbm.at[0, pl.ds(ebase, n_edges)],
                                srcb[0].at[pl.ds(0, n_edges)])
                pltpu.sync_copy(ei_hbm.at[dyn1, pl.ds(ebase, n_edges)],
                                dstb[0].at[pl.ds(0, n_edges)])
                gather_chunk(srcb[0], valb[0], n_edges)
                for d in fire_scatters(valb[0], dstb[0], ssem[0], n_edges):
                    d.wait()
            return tail

        if n_left:
            pl.when(left_id < F)(do_tail(_CHUNK))
        if rem:
            pl.when(left_id == F)(do_tail(rem))

        plsc.subcore_barrier()
        pltpu.sync_copy(acc_s.at[pl.ds(base, words)],
                        out_s_hbm.at[cid, pl.ds(base, words)])
        if with_deg:
            pltpu.sync_copy(acc_d.at[pl.ds(base, words)],
                            out_d_hbm.at[cid, pl.ds(base, words)])

    k = pl.kernel(body, out_type=out_type, mesh=_mesh(),
                  scratch_types=scratch,
                  compiler_params=pltpu.CompilerParams(
                      needs_layout_passes=False))
    return k(table, ei)


def _combine(S2, D2, Wl1, Wr1, b1, Wl2, Wr2, b2):
    NP = S2.shape[1]
    H2 = Wl2.shape[1]

    def body(s2, d2, wl1, wr1, b1r, wl2, wr2, b2r,
             mean1_o, gamma_o, dinv_o, u_o):
        s = s2[0, :] + s2[1, :]
        deg = d2[0, :] + d2[1, :]
        dm = jnp.maximum(deg, 1.0)
        mean1_o[:] = s / dm
        gamma_o[:] = deg / dm
        dinv_o[:] = 1.0 / dm
        u1 = jnp.dot(wl1[:], wl2[:])
        u24 = jnp.dot(wr1[:], wl2[:]) + jnp.dot(wl1[:], wr2[:])
        u3 = jnp.dot(b1r[:][None, :], wl2[:])
        u5 = jnp.dot(wr1[:], wr2[:])
        c0 = jnp.dot(b1r[:][None, :], wr2[:]) + b2r[:][None, :]
        z = jnp.zeros((3, H2), jnp.float32)
        u_o[:, :] = jnp.concatenate([u1, u24, u3, u5, c0, z], axis=0)

    return pl.pallas_call(
        body,
        out_shape=[
            jax.ShapeDtypeStruct((NP,), jnp.float32),
            jax.ShapeDtypeStruct((NP,), jnp.float32),
            jax.ShapeDtypeStruct((NP,), jnp.float32),
            jax.ShapeDtypeStruct((8, H2), jnp.float32),
        ],
    )(S2, D2, Wl1, Wr1, b1, Wl2, Wr2, b2)


@functools.partial(jax.jit, static_argnames=("kpad",))
def _pool(A0, A1, mean1, gamma, dinv, xv, cl, U, *, kpad):
    NP = mean1.shape[0]
    per = NP // _NW

    scratch = [
        pltpu.VMEM((per,), jnp.float32),
        pltpu.VMEM((per,), jnp.float32),
        pltpu.VMEM((per,), jnp.float32),
        pltpu.VMEM((per,), jnp.float32),
        pltpu.VMEM((per,), jnp.float32),
        pltpu.VMEM((per,), jnp.float32),
        pltpu.VMEM((per,), jnp.int32),
        pltpu.VMEM((8, 32), jnp.float32),
        pltpu.VMEM((kpad, 32), jnp.float32),
        pltpu.SemaphoreType.DMA,
    ]

    def body(a0_hbm, a1_hbm, m1_hbm, ga_hbm, di_hbm, x_hbm, cl_hbm, u_hbm,
             out_hbm, a0v, a1v, m1v, gav, div, xvv, clv, uv, pooled, sem):
        cid = lax.axis_index("c")
        sid = lax.axis_index("s")
        w = cid * _NS + sid
        base = w * per
        descs = [
            pltpu.async_copy(a0_hbm.at[pl.ds(base, per)], a0v, sem),
            pltpu.async_copy(a1_hbm.at[pl.ds(base, per)], a1v, sem),
            pltpu.async_copy(m1_hbm.at[pl.ds(base, per)], m1v, sem),
            pltpu.async_copy(ga_hbm.at[pl.ds(base, per)], gav, sem),
            pltpu.async_copy(di_hbm.at[pl.ds(base, per)], div, sem),
            pltpu.async_copy(x_hbm.at[pl.ds(base, per)], xvv, sem),
            pltpu.async_copy(cl_hbm.at[pl.ds(base, per)], clv, sem),
            pltpu.async_copy(u_hbm, uv, sem),
        ]

        ninf = jnp.full((16,), -jnp.inf, jnp.float32)

        def pinit(k, _):
            pooled[k, pl.ds(0, 16)] = ninf
            pooled[k, pl.ds(16, 16)] = ninf
            return 0
        lax.fori_loop(0, kpad, pinit, 0)
        for d in descs:
            d.wait()

        u1a = uv[0, pl.ds(0, 16)]
        u1b = uv[0, pl.ds(16, 16)]
        u24a = uv[1, pl.ds(0, 16)]
        u24b = uv[1, pl.ds(16, 16)]
        u3a = uv[2, pl.ds(0, 16)]
        u3b = uv[2, pl.ds(16, 16)]
        u5a = uv[3, pl.ds(0, 16)]
        u5b = uv[3, pl.ds(16, 16)]

        def nb(i, carry):
            cprev, m0, m1 = carry
            b16 = i * 16
            alv = ((a0v[pl.ds(b16, 16)] + a1v[pl.ds(b16, 16)])
                   * div[pl.ds(b16, 16)])
            bev = m1v[pl.ds(b16, 16)]
            gav16 = gav[pl.ds(b16, 16)]
            xxv = xvv[pl.ds(b16, 16)]
            clv16 = clv[pl.ds(b16, 16)]
            for j in range(16):
                al = alv[j]
                be = bev[j]
                ga = gav16[j]
                xx = xxv[j]
                c = clv16[j]
                h0 = (al * u1a + be * u24a) + (ga * u3a + xx * u5a)
                h1 = (al * u1b + be * u24b) + (ga * u3b + xx * u5b)
                flush = c != cprev

                def do_flush(cp=cprev, a=m0, b=m1):
                    pooled[cp, pl.ds(0, 16)] = a
                    pooled[cp, pl.ds(16, 16)] = b
                    return ninf, ninf

                def no_flush(a=m0, b=m1):
                    return a, b
                m0, m1 = lax.cond(flush, do_flush, no_flush)
                m0 = jnp.maximum(m0, h0)
                m1 = jnp.maximum(m1, h1)
                cprev = c
            return cprev, m0, m1
        cprev, m0, m1 = lax.fori_loop(
            0, per // 16, nb,
            (jnp.int32(kpad - 1), ninf, ninf))
        pooled[cprev, pl.ds(0, 16)] = m0
        pooled[cprev, pl.ds(16, 16)] = m1

        pltpu.sync_copy(pooled, out_hbm.at[w])

    k = pl.kernel(
        body,
        out_type=jax.ShapeDtypeStruct((_NW, kpad, 32), jnp.float32),
        mesh=_mesh(),
        scratch_types=scratch,
        compiler_params=pltpu.CompilerParams(needs_layout_passes=False),
    )
    return k(A0, A1, mean1, gamma, dinv, xv, cl, U)


def _head(p32, nclusters, We3, be, Wp1, bp1, Wp2, bp2, U):
    def body(p, we3, ber, wp1, bp1r, wp2, bp2r, u, out):
        pm = jnp.max(p[...], axis=0)
        pm = pm[:nclusters, :]
        pm = jnp.where(jnp.isfinite(pm), pm + u[4, :][None, :], 0.0)
        t = pm[:, :, None] * we3[...]
        emb = jnp.sum(jnp.sum(t, axis=0), axis=0, keepdims=True)
        emb = jnp.maximum(emb + ber[:][None, :], 0.0)
        p1 = jnp.maximum(jnp.dot(emb, wp1[...]) + bp1r[:][None, :], 0.0)
        out[:, :] = jax.nn.sigmoid(jnp.dot(p1, wp2[...]) + bp2r[:][None, :])

    return pl.pallas_call(
        body,
        out_shape=jax.ShapeDtypeStruct((1, 1), jnp.float32),
    )(p32, We3, be, Wp1, bp1, Wp2, bp2, U)


def kernel(x, edge_index, clusters, Wl1, Wr1, b1, Wl2, Wr2, b2,
           We, be, Wp1, bp1, Wp2, bp2):
    N = x.shape[0]
    H2 = Wl2.shape[1]
    K = We.shape[0] // H2
    R = We.shape[1]

    NP = ((N + 2047) // 2048) * 2048
    kpad = ((K + 1 + 15) // 16) * 16

    xv = x[:, 0]
    xp = jnp.concatenate([xv, jnp.zeros((NP - N,), jnp.float32)])
    clp = jnp.concatenate(
        [clusters, jnp.full((NP - N,), K, jnp.int32)])

    S2, D2 = _edge_pass(xv, edge_index, with_deg=True, NP=NP)
    mean1, gamma, dinv, U = _combine(S2, D2, Wl1, Wr1, b1, Wl2, Wr2, b2)
    (A2,) = _edge_pass(mean1, edge_index, with_deg=False, NP=NP)
    p32 = _pool(A2[0], A2[1], mean1, gamma, dinv, xp, clp, U, kpad=kpad)
    We3 = We.reshape(K, H2, R)
    return _head(p32, K, We3, be, Wp1, bp1, Wp2, bp2, U)


# speedup vs baseline: 1.0540x; 1.0099x over previous
import functools

import jax
import jax.numpy as jnp
from jax import lax
from jax.experimental import pallas as pl
from jax.experimental.pallas import tpu as pltpu
from jax.experimental.pallas import tpu_sc as plsc

_NC = 2
_NS = 16
_NW = _NC * _NS
_CHUNK = 2048
_ROWS = _CHUNK // 128


def _mesh():
    return plsc.VectorSubcoreMesh(core_axis_name="c", subcore_axis_name="s",
                                  num_cores=_NC, num_subcores=_NS)


@functools.partial(jax.jit, static_argnames=("with_deg", "NP"))
def _edge_pass(table, ei, *, with_deg, NP):
    TN = table.shape[0]
    E = ei.shape[1]
    F, rem = divmod(E, _CHUNK)
    assert rem % 128 == 0, "edge count must be a multiple of 128"
    n_pipe = F // _NW
    n_left = F - n_pipe * _NW
    words = NP // _NS

    out_type = [jax.ShapeDtypeStruct((_NC, NP), jnp.float32)]
    if with_deg:
        out_type.append(jax.ShapeDtypeStruct((_NC, NP), jnp.float32))

    NB = 2
    scratch = (
        [pltpu.VMEM((TN,), jnp.float32)]
        + [pltpu.VMEM((_CHUNK,), jnp.int32) for _ in range(NB)]
        + [pltpu.VMEM((_CHUNK,), jnp.int32) for _ in range(NB)]
        + [pltpu.VMEM((_CHUNK,), jnp.float32) for _ in range(NB)]
        + [pltpu.VMEM((_CHUNK,), jnp.float32),
           pltpu.VMEM_SHARED((NP,), jnp.float32)]
        + ([pltpu.VMEM_SHARED((NP,), jnp.float32)] if with_deg else [])
        + [pltpu.SemaphoreType.DMA for _ in range(2 * NB)]
    )

    def body(table_hbm, ei_hbm, *rest):
        rest = list(rest)
        out_s_hbm = rest.pop(0)
        out_d_hbm = rest.pop(0) if with_deg else None
        tbl = rest.pop(0)
        srcb = [rest.pop(0) for _ in range(NB)]
        dstb = [rest.pop(0) for _ in range(NB)]
        valb = [rest.pop(0) for _ in range(NB)]
        util = rest.pop(0)
        acc_s = rest.pop(0)
        acc_d = rest.pop(0) if with_deg else None
        lsem = [rest.pop(0) for _ in range(NB)]
        ssem = [rest.pop(0) for _ in range(NB)]
        assert not rest

        cid = lax.axis_index("c")
        sid = lax.axis_index("s")
        dyn1 = jnp.minimum(cid, 0) + 1

        def fillz(i, _):
            util[pl.ds(i * 16, 16)] = jnp.zeros((16,), jnp.float32)
            return 0
        lax.fori_loop(0, _CHUNK // 16, fillz, 0)

        base = sid * words
        n_full_z, rem_z = divmod(words, _CHUNK)
        pro = [pltpu.async_copy(table_hbm, tbl, ssem[0])]
        for k in range(n_full_z):
            pro.append(pltpu.async_copy(
                util, acc_s.at[pl.ds(base + k * _CHUNK, _CHUNK)], ssem[1]))
            if with_deg:
                pro.append(pltpu.async_copy(
                    util, acc_d.at[pl.ds(base + k * _CHUNK, _CHUNK)],
                    ssem[1]))
        if rem_z:
            pro.append(pltpu.async_copy(
                util.at[pl.ds(0, rem_z)],
                acc_s.at[pl.ds(base + n_full_z * _CHUNK, rem_z)], ssem[1]))
            if with_deg:
                pro.append(pltpu.async_copy(
                    util.at[pl.ds(0, rem_z)],
                    acc_d.at[pl.ds(base + n_full_z * _CHUNK, rem_z)],
                    ssem[1]))
        for d in pro:
            d.wait()

        def fillo(i, _):
            util[pl.ds(i * 16, 16)] = jnp.full((16,), 1.0, jnp.float32)
            return 0
        lax.fori_loop(0, _CHUNK // 16, fillo, 0)
        plsc.subcore_barrier()

        w = cid * _NS + sid

        def gather_chunk(sv, vv, n_edges):
            def g(i, sv=sv, vv=vv):
                vv[pl.ds(i, 16)] = plsc.load_gather(tbl, [sv[pl.ds(i, 16)]])
            plsc.parallel_loop(0, n_edges, step=16, unroll=8)(g)

        def fire_scatters(vv, dv, sem, n_edges):
            idx = dv.at[pl.ds(0, n_edges)]
            cps = [pltpu.async_copy(
                vv.at[pl.ds(0, n_edges)], acc_s.at[idx], sem, add=True)]
            if with_deg:
                cps.append(pltpu.async_copy(
                    util.at[pl.ds(0, n_edges)], acc_d.at[idx],
                    sem, add=True))
            return cps

        load_descs = {}
        scat_descs = {}

        def start_loads(ci):
            chunk_id = w * n_pipe + ci
            b = ci % NB
            d1 = pltpu.async_copy(
                ei_hbm.at[0, pl.ds(chunk_id * _CHUNK, _CHUNK)], srcb[b],
                lsem[b])
            d2 = pltpu.async_copy(
                ei_hbm.at[dyn1, pl.ds(chunk_id * _CHUNK, _CHUNK)], dstb[b],
                lsem[b])
            load_descs[ci] = [d1, d2]

        start_loads(0)
        for ci in range(n_pipe):
            b = ci % NB
            for d in load_descs.pop(ci):
                d.wait()
            if ci + 1 < n_pipe:
                if ci - 1 >= 0:
                    for d in scat_descs.pop(ci - 1):
                        d.wait()
                start_loads(ci + 1)
            gather_chunk(srcb[b], valb[b], _CHUNK)
            scat_descs[ci] = fire_scatters(valb[b], dstb[b], ssem[b], _CHUNK)
        for ci in sorted(scat_descs):
            for d in scat_descs[ci]:
                d.wait()

        left_id = n_pipe * _NW + w

        def do_tail(n_edges):
            def tail():
                ebase = left_id * _CHUNK
                pltpu.sync_copy(ei_hbm.at[0, pl.ds(ebase, n_edges)],
                                srcb[0].at[pl.ds(0, n_edges)])
                pltpu.sync_copy(ei_hbm.at[dyn1, pl.ds(ebase, n_edges)],
                                dstb[0].at[pl.ds(0, n_edges)])
                gather_chunk(srcb[0], valb[0], n_edges)
                for d in fire_scatters(valb[0], dstb[0], ssem[0], n_edges):
                    d.wait()
            return tail

        if n_left:
            pl.when(left_id < F)(do_tail(_CHUNK))
        if rem:
            pl.when(left_id == F)(do_tail(rem))

        plsc.subcore_barrier()
        pltpu.sync_copy(acc_s.at[pl.ds(base, words)],
                        out_s_hbm.at[cid, pl.ds(base, words)])
        if with_deg:
            pltpu.sync_copy(acc_d.at[pl.ds(base, words)],
                            out_d_hbm.at[cid, pl.ds(base, words)])

    k = pl.kernel(body, out_type=out_type, mesh=_mesh(),
                  scratch_types=scratch,
                  compiler_params=pltpu.CompilerParams(
                      needs_layout_passes=False))
    return k(table, ei)


def _combine(S2, D2, Wl1, Wr1, b1, Wl2, Wr2, b2):
    NP = S2.shape[1]
    H2 = Wl2.shape[1]

    def body(s2, d2, wl1, wr1, b1r, wl2, wr2, b2r,
             mean1_o, gamma_o, dinv_o, u_o):
        s = s2[0, :] + s2[1, :]
        deg = d2[0, :] + d2[1, :]
        dm = jnp.maximum(deg, 1.0)
        mean1_o[:] = s / dm
        gamma_o[:] = deg / dm
        dinv_o[:] = 1.0 / dm
        u1 = jnp.dot(wl1[:], wl2[:])
        u24 = jnp.dot(wr1[:], wl2[:]) + jnp.dot(wl1[:], wr2[:])
        u3 = jnp.dot(b1r[:][None, :], wl2[:])
        u5 = jnp.dot(wr1[:], wr2[:])
        c0 = jnp.dot(b1r[:][None, :], wr2[:]) + b2r[:][None, :]
        z = jnp.zeros((3, H2), jnp.float32)
        u_o[:, :] = jnp.concatenate([u1, u24, u3, u5, c0, z], axis=0)

    return pl.pallas_call(
        body,
        out_shape=[
            jax.ShapeDtypeStruct((NP,), jnp.float32),
            jax.ShapeDtypeStruct((NP,), jnp.float32),
            jax.ShapeDtypeStruct((NP,), jnp.float32),
            jax.ShapeDtypeStruct((8, H2), jnp.float32),
        ],
    )(S2, D2, Wl1, Wr1, b1, Wl2, Wr2, b2)


@functools.partial(jax.jit, static_argnames=("kpad",))
def _pool(A0, A1, mean1, gamma, dinv, xv, cl, U, *, kpad):
    NP = mean1.shape[0]
    per = NP // _NW

    scratch = [
        pltpu.VMEM((per,), jnp.float32),
        pltpu.VMEM((per,), jnp.float32),
        pltpu.VMEM((per,), jnp.float32),
        pltpu.VMEM((per,), jnp.float32),
        pltpu.VMEM((per,), jnp.float32),
        pltpu.VMEM((per,), jnp.float32),
        pltpu.VMEM((per,), jnp.int32),
        pltpu.VMEM((8, 32), jnp.float32),
        pltpu.VMEM((kpad, 32), jnp.float32),
        pltpu.SemaphoreType.DMA,
    ]

    def body(a0_hbm, a1_hbm, m1_hbm, ga_hbm, di_hbm, x_hbm, cl_hbm, u_hbm,
             out_hbm, a0v, a1v, m1v, gav, div, xvv, clv, uv, pooled, sem):
        cid = lax.axis_index("c")
        sid = lax.axis_index("s")
        w = cid * _NS + sid
        base = w * per
        descs = [
            pltpu.async_copy(a0_hbm.at[pl.ds(base, per)], a0v, sem),
            pltpu.async_copy(a1_hbm.at[pl.ds(base, per)], a1v, sem),
            pltpu.async_copy(m1_hbm.at[pl.ds(base, per)], m1v, sem),
            pltpu.async_copy(ga_hbm.at[pl.ds(base, per)], gav, sem),
            pltpu.async_copy(di_hbm.at[pl.ds(base, per)], div, sem),
            pltpu.async_copy(x_hbm.at[pl.ds(base, per)], xvv, sem),
            pltpu.async_copy(cl_hbm.at[pl.ds(base, per)], clv, sem),
            pltpu.async_copy(u_hbm, uv, sem),
        ]

        ninf = jnp.full((16,), -jnp.inf, jnp.float32)

        def pinit(k, _):
            pooled[k, pl.ds(0, 16)] = ninf
            pooled[k, pl.ds(16, 16)] = ninf
            return 0
        lax.fori_loop(0, kpad, pinit, 0)
        for d in descs:
            d.wait()

        u1a = uv[0, pl.ds(0, 16)]
        u1b = uv[0, pl.ds(16, 16)]
        u24a = uv[1, pl.ds(0, 16)]
        u24b = uv[1, pl.ds(16, 16)]
        u3a = uv[2, pl.ds(0, 16)]
        u3b = uv[2, pl.ds(16, 16)]
        u5a = uv[3, pl.ds(0, 16)]
        u5b = uv[3, pl.ds(16, 16)]

        def nb(i, carry):
            cprev, m0, m1 = carry
            b16 = i * 16
            alv = ((a0v[pl.ds(b16, 16)] + a1v[pl.ds(b16, 16)])
                   * div[pl.ds(b16, 16)])
            bev = m1v[pl.ds(b16, 16)]
            gav16 = gav[pl.ds(b16, 16)]
            xxv = xvv[pl.ds(b16, 16)]
            clv16 = clv[pl.ds(b16, 16)]
            for j in range(16):
                al = alv[j]
                be = bev[j]
                ga = gav16[j]
                xx = xxv[j]
                c = clv16[j]
                h0 = (al * u1a + be * u24a) + (ga * u3a + xx * u5a)
                h1 = (al * u1b + be * u24b) + (ga * u3b + xx * u5b)
                flush = c != cprev

                def do_flush(cp=cprev, a=m0, b=m1):
                    pooled[cp, pl.ds(0, 16)] = a
                    pooled[cp, pl.ds(16, 16)] = b
                    return ninf, ninf

                def no_flush(a=m0, b=m1):
                    return a, b
                m0, m1 = lax.cond(flush, do_flush, no_flush)
                m0 = jnp.maximum(m0, h0)
                m1 = jnp.maximum(m1, h1)
                cprev = c
            return cprev, m0, m1
        cprev, m0, m1 = lax.fori_loop(
            0, per // 16, nb,
            (jnp.int32(kpad - 1), ninf, ninf))
        pooled[cprev, pl.ds(0, 16)] = m0
        pooled[cprev, pl.ds(16, 16)] = m1

        pltpu.sync_copy(pooled, out_hbm.at[w])

    k = pl.kernel(
        body,
        out_type=jax.ShapeDtypeStruct((_NW, kpad, 32), jnp.float32),
        mesh=_mesh(),
        scratch_types=scratch,
        compiler_params=pltpu.CompilerParams(needs_layout_passes=False),
    )
    return k(A0, A1, mean1, gamma, dinv, xv, cl, U)


def _head(p32, nclusters, We3, be, Wp1, bp1, Wp2, bp2, U):
    def body(p, we3, ber, wp1, bp1r, wp2, bp2r, u, out):
        pm = jnp.max(p[...], axis=0)
        pm = pm[:nclusters, :]
        pm = jnp.where(jnp.isfinite(pm), pm + u[4, :][None, :], 0.0)
        t = pm[:, :, None] * we3[...]
        emb = jnp.sum(jnp.sum(t, axis=0), axis=0, keepdims=True)
        emb = jnp.maximum(emb + ber[:][None, :], 0.0)
        p1 = jnp.maximum(jnp.dot(emb, wp1[...]) + bp1r[:][None, :], 0.0)
        out[:, :] = jax.nn.sigmoid(jnp.dot(p1, wp2[...]) + bp2r[:][None, :])

    return pl.pallas_call(
        body,
        out_shape=jax.ShapeDtypeStruct((1, 1), jnp.float32),
    )(p32, We3, be, Wp1, bp1, Wp2, bp2, U)


def kernel(x, edge_index, clusters, Wl1, Wr1, b1, Wl2, Wr2, b2,
           We, be, Wp1, bp1, Wp2, bp2):
    N = x.shape[0]
    H2 = Wl2.shape[1]
    K = We.shape[0] // H2
    R = We.shape[1]

    NP = ((N + 2047) // 2048) * 2048
    kpad = ((K + 1 + 15) // 16) * 16

    xv = x[:, 0]
    xp = jnp.concatenate([xv, jnp.zeros((NP - N,), jnp.float32)])
    clp = jnp.concatenate(
        [clusters, jnp.full((NP - N,), K, jnp.int32)])

    S2, D2 = _edge_pass(xv, edge_index, with_deg=True, NP=NP)
    mean1, gamma, dinv, U = _combine(S2, D2, Wl1, Wr1, b1, Wl2, Wr2, b2)
    (A2,) = _edge_pass(mean1, edge_index, with_deg=False, NP=NP)
    p32 = _pool(A2[0], A2[1], mean1, gamma, dinv, xp, clp, U, kpad=kpad)
    We3 = We.reshape(K, H2, R)
    return _head(p32, K, We3, be, Wp1, bp1, Wp2, bp2, U)


# speedup vs baseline: 1.1175x; 1.0603x over previous
import functools

import jax
import jax.numpy as jnp
from jax import lax
from jax.experimental import pallas as pl
from jax.experimental.pallas import tpu as pltpu
from jax.experimental.pallas import tpu_sc as plsc

_NC = 2
_NS = 16
_NW = _NC * _NS
_CHUNK = 2048
_ROWS = _CHUNK // 128


def _mesh():
    return plsc.VectorSubcoreMesh(core_axis_name="c", subcore_axis_name="s",
                                  num_cores=_NC, num_subcores=_NS)


@functools.partial(jax.jit, static_argnames=("with_deg", "NP"))
def _edge_pass(table, ei, *, with_deg, NP):
    TN = table.shape[0]
    E = ei.shape[1]
    F, rem = divmod(E, _CHUNK)
    assert rem % 128 == 0, "edge count must be a multiple of 128"
    n_pipe = F // _NW
    n_left = F - n_pipe * _NW
    words = NP // _NS

    out_type = [jax.ShapeDtypeStruct((_NC, NP), jnp.float32)]
    if with_deg:
        out_type.append(jax.ShapeDtypeStruct((_NC, NP), jnp.float32))

    NS_, ND_ = 2, 3
    scratch = (
        [pltpu.VMEM((TN,), jnp.float32)]
        + [pltpu.VMEM((_CHUNK,), jnp.int32) for _ in range(NS_)]
        + [pltpu.VMEM((_CHUNK,), jnp.int32) for _ in range(ND_)]
        + [pltpu.VMEM((_CHUNK,), jnp.float32) for _ in range(ND_)]
        + [pltpu.VMEM((_CHUNK,), jnp.float32),
           pltpu.VMEM_SHARED((NP,), jnp.float32)]
        + ([pltpu.VMEM_SHARED((NP,), jnp.float32)] if with_deg else [])
        + [pltpu.SemaphoreType.DMA for _ in range(NS_ + 2 * ND_)]
    )

    def body(table_hbm, ei_hbm, *rest):
        rest = list(rest)
        out_s_hbm = rest.pop(0)
        out_d_hbm = rest.pop(0) if with_deg else None
        tbl = rest.pop(0)
        srcb = [rest.pop(0) for _ in range(NS_)]
        dstb = [rest.pop(0) for _ in range(ND_)]
        valb = [rest.pop(0) for _ in range(ND_)]
        util = rest.pop(0)
        acc_s = rest.pop(0)
        acc_d = rest.pop(0) if with_deg else None
        lsem = [rest.pop(0) for _ in range(NS_)]
        dsem = [rest.pop(0) for _ in range(ND_)]
        ssem = [rest.pop(0) for _ in range(ND_)]
        assert not rest

        cid = lax.axis_index("c")
        sid = lax.axis_index("s")
        dyn1 = jnp.minimum(cid, 0) + 1

        def fillz(i, _):
            util[pl.ds(i * 16, 16)] = jnp.zeros((16,), jnp.float32)
            return 0
        lax.fori_loop(0, _CHUNK // 16, fillz, 0)

        base = sid * words
        n_full_z, rem_z = divmod(words, _CHUNK)
        pro = [pltpu.async_copy(table_hbm, tbl, ssem[0])]
        for k in range(n_full_z):
            pro.append(pltpu.async_copy(
                util, acc_s.at[pl.ds(base + k * _CHUNK, _CHUNK)], ssem[1]))
            if with_deg:
                pro.append(pltpu.async_copy(
                    util, acc_d.at[pl.ds(base + k * _CHUNK, _CHUNK)],
                    ssem[1]))
        if rem_z:
            pro.append(pltpu.async_copy(
                util.at[pl.ds(0, rem_z)],
                acc_s.at[pl.ds(base + n_full_z * _CHUNK, rem_z)], ssem[1]))
            if with_deg:
                pro.append(pltpu.async_copy(
                    util.at[pl.ds(0, rem_z)],
                    acc_d.at[pl.ds(base + n_full_z * _CHUNK, rem_z)],
                    ssem[1]))
        for d in pro:
            d.wait()

        def fillo(i, _):
            util[pl.ds(i * 16, 16)] = jnp.full((16,), 1.0, jnp.float32)
            return 0
        lax.fori_loop(0, _CHUNK // 16, fillo, 0)
        plsc.subcore_barrier()

        w = cid * _NS + sid

        def gather_chunk(sv, vv, n_edges):
            def g(i, sv=sv, vv=vv):
                vv[pl.ds(i, 16)] = plsc.load_gather(tbl, [sv[pl.ds(i, 16)]])
            plsc.parallel_loop(0, n_edges, step=16, unroll=8)(g)

        def fire_scatters(vv, dv, sem, n_edges):
            idx = dv.at[pl.ds(0, n_edges)]
            cps = [pltpu.async_copy(
                vv.at[pl.ds(0, n_edges)], acc_s.at[idx], sem, add=True)]
            if with_deg:
                cps.append(pltpu.async_copy(
                    util.at[pl.ds(0, n_edges)], acc_d.at[idx],
                    sem, add=True))
            return cps

        load_descs = {}
        scat_descs = {}

        def start_loads(ci):
            chunk_id = w * n_pipe + ci
            s, b = ci % NS_, ci % ND_
            d1 = pltpu.async_copy(
                ei_hbm.at[0, pl.ds(chunk_id * _CHUNK, _CHUNK)], srcb[s],
                lsem[s])
            d2 = pltpu.async_copy(
                ei_hbm.at[dyn1, pl.ds(chunk_id * _CHUNK, _CHUNK)], dstb[b],
                dsem[b])
            load_descs[ci] = [d1, d2]

        start_loads(0)
        for ci in range(n_pipe):
            s, b = ci % NS_, ci % ND_
            for d in load_descs.pop(ci):
                d.wait()
            if ci + 1 < n_pipe:
                if ci - 2 >= 0:
                    for d in scat_descs.pop(ci - 2):
                        d.wait()
                start_loads(ci + 1)
            gather_chunk(srcb[s], valb[b], _CHUNK)
            scat_descs[ci] = fire_scatters(valb[b], dstb[b], ssem[b], _CHUNK)
        for ci in sorted(scat_descs):
            for d in scat_descs[ci]:
                d.wait()

        left_id = n_pipe * _NW + w

        def do_tail(n_edges):
            def tail():
                ebase = left_id * _CHUNK
                pltpu.sync_copy(ei_hbm.at[0, pl.ds(ebase, n_edges)],
                                srcb[0].at[pl.ds(0, n_edges)])
                pltpu.sync_copy(ei_hbm.at[dyn1, pl.ds(ebase, n_edges)],
                                dstb[0].at[pl.ds(0, n_edges)])
                gather_chunk(srcb[0], valb[0], n_edges)
                for d in fire_scatters(valb[0], dstb[0], ssem[0], n_edges):
                    d.wait()
            return tail

        if n_left:
            pl.when(left_id < F)(do_tail(_CHUNK))
        if rem:
            pl.when(left_id == F)(do_tail(rem))

        plsc.subcore_barrier()
        pltpu.sync_copy(acc_s.at[pl.ds(base, words)],
                        out_s_hbm.at[cid, pl.ds(base, words)])
        if with_deg:
            pltpu.sync_copy(acc_d.at[pl.ds(base, words)],
                            out_d_hbm.at[cid, pl.ds(base, words)])

    k = pl.kernel(body, out_type=out_type, mesh=_mesh(),
                  scratch_types=scratch,
                  compiler_params=pltpu.CompilerParams(
                      needs_layout_passes=False))
    return k(table, ei)


def _combine(S2, D2, Wl1, Wr1, b1, Wl2, Wr2, b2):
    NP = S2.shape[1]
    H2 = Wl2.shape[1]

    def body(s2, d2, wl1, wr1, b1r, wl2, wr2, b2r,
             mean1_o, gamma_o, dinv_o, u_o):
        s = s2[0, :] + s2[1, :]
        deg = d2[0, :] + d2[1, :]
        dm = jnp.maximum(deg, 1.0)
        mean1_o[:] = s / dm
        gamma_o[:] = deg / dm
        dinv_o[:] = 1.0 / dm
        u1 = jnp.dot(wl1[:], wl2[:])
        u24 = jnp.dot(wr1[:], wl2[:]) + jnp.dot(wl1[:], wr2[:])
        u3 = jnp.dot(b1r[:][None, :], wl2[:])
        u5 = jnp.dot(wr1[:], wr2[:])
        c0 = jnp.dot(b1r[:][None, :], wr2[:]) + b2r[:][None, :]
        z = jnp.zeros((3, H2), jnp.float32)
        u_o[:, :] = jnp.concatenate([u1, u24, u3, u5, c0, z], axis=0)

    return pl.pallas_call(
        body,
        out_shape=[
            jax.ShapeDtypeStruct((NP,), jnp.float32),
            jax.ShapeDtypeStruct((NP,), jnp.float32),
            jax.ShapeDtypeStruct((NP,), jnp.float32),
            jax.ShapeDtypeStruct((8, H2), jnp.float32),
        ],
    )(S2, D2, Wl1, Wr1, b1, Wl2, Wr2, b2)


@functools.partial(jax.jit, static_argnames=("kpad",))
def _pool(A0, A1, mean1, gamma, dinv, xv, cl, U, *, kpad):
    NP = mean1.shape[0]
    per = NP // _NW

    scratch = [
        pltpu.VMEM((per,), jnp.float32),
        pltpu.VMEM((per,), jnp.float32),
        pltpu.VMEM((per,), jnp.float32),
        pltpu.VMEM((per,), jnp.float32),
        pltpu.VMEM((per,), jnp.float32),
        pltpu.VMEM((per,), jnp.float32),
        pltpu.VMEM((per,), jnp.int32),
        pltpu.VMEM((8, 32), jnp.float32),
        pltpu.VMEM((kpad, 32), jnp.float32),
        pltpu.SemaphoreType.DMA,
    ]

    def body(a0_hbm, a1_hbm, m1_hbm, ga_hbm, di_hbm, x_hbm, cl_hbm, u_hbm,
             out_hbm, a0v, a1v, m1v, gav, div, xvv, clv, uv, pooled, sem):
        cid = lax.axis_index("c")
        sid = lax.axis_index("s")
        w = cid * _NS + sid
        base = w * per
        descs = [
            pltpu.async_copy(a0_hbm.at[pl.ds(base, per)], a0v, sem),
            pltpu.async_copy(a1_hbm.at[pl.ds(base, per)], a1v, sem),
            pltpu.async_copy(m1_hbm.at[pl.ds(base, per)], m1v, sem),
            pltpu.async_copy(ga_hbm.at[pl.ds(base, per)], gav, sem),
            pltpu.async_copy(di_hbm.at[pl.ds(base, per)], div, sem),
            pltpu.async_copy(x_hbm.at[pl.ds(base, per)], xvv, sem),
            pltpu.async_copy(cl_hbm.at[pl.ds(base, per)], clv, sem),
            pltpu.async_copy(u_hbm, uv, sem),
        ]

        ninf = jnp.full((16,), -jnp.inf, jnp.float32)

        def pinit(k, _):
            pooled[k, pl.ds(0, 16)] = ninf
            pooled[k, pl.ds(16, 16)] = ninf
            return 0
        lax.fori_loop(0, kpad, pinit, 0)
        for d in descs:
            d.wait()

        u1a = uv[0, pl.ds(0, 16)]
        u1b = uv[0, pl.ds(16, 16)]
        u24a = uv[1, pl.ds(0, 16)]
        u24b = uv[1, pl.ds(16, 16)]
        u3a = uv[2, pl.ds(0, 16)]
        u3b = uv[2, pl.ds(16, 16)]
        u5a = uv[3, pl.ds(0, 16)]
        u5b = uv[3, pl.ds(16, 16)]

        def nb(i, carry):
            cprev, m0, m1 = carry
            b16 = i * 16
            alv = ((a0v[pl.ds(b16, 16)] + a1v[pl.ds(b16, 16)])
                   * div[pl.ds(b16, 16)])
            bev = m1v[pl.ds(b16, 16)]
            gav16 = gav[pl.ds(b16, 16)]
            xxv = xvv[pl.ds(b16, 16)]
            clv16 = clv[pl.ds(b16, 16)]
            for j in range(16):
                al = alv[j]
                be = bev[j]
                ga = gav16[j]
                xx = xxv[j]
                c = clv16[j]
                h0 = (al * u1a + be * u24a) + (ga * u3a + xx * u5a)
                h1 = (al * u1b + be * u24b) + (ga * u3b + xx * u5b)
                flush = c != cprev

                def do_flush(cp=cprev, a=m0, b=m1):
                    pooled[cp, pl.ds(0, 16)] = a
                    pooled[cp, pl.ds(16, 16)] = b
                    return ninf, ninf

                def no_flush(a=m0, b=m1):
                    return a, b
                m0, m1 = lax.cond(flush, do_flush, no_flush)
                m0 = jnp.maximum(m0, h0)
                m1 = jnp.maximum(m1, h1)
                cprev = c
            return cprev, m0, m1
        cprev, m0, m1 = lax.fori_loop(
            0, per // 16, nb,
            (jnp.int32(kpad - 1), ninf, ninf))
        pooled[cprev, pl.ds(0, 16)] = m0
        pooled[cprev, pl.ds(16, 16)] = m1

        pltpu.sync_copy(pooled, out_hbm.at[w])

    k = pl.kernel(
        body,
        out_type=jax.ShapeDtypeStruct((_NW, kpad, 32), jnp.float32),
        mesh=_mesh(),
        scratch_types=scratch,
        compiler_params=pltpu.CompilerParams(needs_layout_passes=False),
    )
    return k(A0, A1, mean1, gamma, dinv, xv, cl, U)


def _head(p32, nclusters, We3, be, Wp1, bp1, Wp2, bp2, U):
    def body(p, we3, ber, wp1, bp1r, wp2, bp2r, u, out):
        pm = jnp.max(p[...], axis=0)
        pm = pm[:nclusters, :]
        pm = jnp.where(jnp.isfinite(pm), pm + u[4, :][None, :], 0.0)
        t = pm[:, :, None] * we3[...]
        emb = jnp.sum(jnp.sum(t, axis=0), axis=0, keepdims=True)
        emb = jnp.maximum(emb + ber[:][None, :], 0.0)
        p1 = jnp.maximum(jnp.dot(emb, wp1[...]) + bp1r[:][None, :], 0.0)
        out[:, :] = jax.nn.sigmoid(jnp.dot(p1, wp2[...]) + bp2r[:][None, :])

    return pl.pallas_call(
        body,
        out_shape=jax.ShapeDtypeStruct((1, 1), jnp.float32),
    )(p32, We3, be, Wp1, bp1, Wp2, bp2, U)


def kernel(x, edge_index, clusters, Wl1, Wr1, b1, Wl2, Wr2, b2,
           We, be, Wp1, bp1, Wp2, bp2):
    N = x.shape[0]
    H2 = Wl2.shape[1]
    K = We.shape[0] // H2
    R = We.shape[1]

    NP = ((N + 2047) // 2048) * 2048
    kpad = ((K + 1 + 15) // 16) * 16

    xv = x[:, 0]
    xp = jnp.concatenate([xv, jnp.zeros((NP - N,), jnp.float32)])
    clp = jnp.concatenate(
        [clusters, jnp.full((NP - N,), K, jnp.int32)])

    S2, D2 = _edge_pass(xv, edge_index, with_deg=True, NP=NP)
    mean1, gamma, dinv, U = _combine(S2, D2, Wl1, Wr1, b1, Wl2, Wr2, b2)
    (A2,) = _edge_pass(mean1, edge_index, with_deg=False, NP=NP)
    p32 = _pool(A2[0], A2[1], mean1, gamma, dinv, xp, clp, U, kpad=kpad)
    We3 = We.reshape(K, H2, R)
    return _head(p32, K, We3, be, Wp1, bp1, Wp2, bp2, U)


# speedup vs baseline: 1.1184x; 1.0008x over previous
import functools

import jax
import jax.numpy as jnp
from jax import lax
from jax.experimental import pallas as pl
from jax.experimental.pallas import tpu as pltpu
from jax.experimental.pallas import tpu_sc as plsc

_NC = 2
_NS = 16
_NW = _NC * _NS
_CHUNK = 2048
_ROWS = _CHUNK // 128


def _mesh():
    return plsc.VectorSubcoreMesh(core_axis_name="c", subcore_axis_name="s",
                                  num_cores=_NC, num_subcores=_NS)


@functools.partial(jax.jit, static_argnames=("with_deg", "NP"))
def _edge_pass(table, ei, *, with_deg, NP):
    TN = table.shape[0]
    E = ei.shape[1]
    F, rem = divmod(E, _CHUNK)
    assert rem % 128 == 0, "edge count must be a multiple of 128"
    n_pipe = F // _NW
    n_left = F - n_pipe * _NW
    words = NP // _NS

    out_type = [jax.ShapeDtypeStruct((_NC, NP), jnp.float32)]
    if with_deg:
        out_type.append(jax.ShapeDtypeStruct((_NC, NP), jnp.float32))

    NS_, ND_ = 2, (3 if with_deg else 4)
    scratch = (
        [pltpu.VMEM((TN,), jnp.float32)]
        + [pltpu.VMEM((_CHUNK,), jnp.int32) for _ in range(NS_)]
        + [pltpu.VMEM((_CHUNK,), jnp.int32) for _ in range(ND_)]
        + [pltpu.VMEM((_CHUNK,), jnp.float32) for _ in range(ND_)]
        + [pltpu.VMEM((_CHUNK,), jnp.float32),
           pltpu.VMEM_SHARED((NP,), jnp.float32)]
        + ([pltpu.VMEM_SHARED((NP,), jnp.float32)] if with_deg else [])
        + [pltpu.SemaphoreType.DMA for _ in range(NS_ + 2 * ND_)]
    )

    def body(table_hbm, ei_hbm, *rest):
        rest = list(rest)
        out_s_hbm = rest.pop(0)
        out_d_hbm = rest.pop(0) if with_deg else None
        tbl = rest.pop(0)
        srcb = [rest.pop(0) for _ in range(NS_)]
        dstb = [rest.pop(0) for _ in range(ND_)]
        valb = [rest.pop(0) for _ in range(ND_)]
        util = rest.pop(0)
        acc_s = rest.pop(0)
        acc_d = rest.pop(0) if with_deg else None
        lsem = [rest.pop(0) for _ in range(NS_)]
        dsem = [rest.pop(0) for _ in range(ND_)]
        ssem = [rest.pop(0) for _ in range(ND_)]
        assert not rest

        cid = lax.axis_index("c")
        sid = lax.axis_index("s")
        dyn1 = jnp.minimum(cid, 0) + 1

        def fillz(i, _):
            util[pl.ds(i * 16, 16)] = jnp.zeros((16,), jnp.float32)
            return 0
        lax.fori_loop(0, _CHUNK // 16, fillz, 0)

        base = sid * words
        n_full_z, rem_z = divmod(words, _CHUNK)
        pro = [pltpu.async_copy(table_hbm, tbl, ssem[0])]
        for k in range(n_full_z):
            pro.append(pltpu.async_copy(
                util, acc_s.at[pl.ds(base + k * _CHUNK, _CHUNK)], ssem[1]))
            if with_deg:
                pro.append(pltpu.async_copy(
                    util, acc_d.at[pl.ds(base + k * _CHUNK, _CHUNK)],
                    ssem[1]))
        if rem_z:
            pro.append(pltpu.async_copy(
                util.at[pl.ds(0, rem_z)],
                acc_s.at[pl.ds(base + n_full_z * _CHUNK, rem_z)], ssem[1]))
            if with_deg:
                pro.append(pltpu.async_copy(
                    util.at[pl.ds(0, rem_z)],
                    acc_d.at[pl.ds(base + n_full_z * _CHUNK, rem_z)],
                    ssem[1]))
        for d in pro:
            d.wait()

        def fillo(i, _):
            util[pl.ds(i * 16, 16)] = jnp.full((16,), 1.0, jnp.float32)
            return 0
        lax.fori_loop(0, _CHUNK // 16, fillo, 0)
        plsc.subcore_barrier()

        w = cid * _NS + sid

        def gather_chunk(sv, vv, n_edges):
            def g(i, sv=sv, vv=vv):
                vv[pl.ds(i, 16)] = plsc.load_gather(tbl, [sv[pl.ds(i, 16)]])
            plsc.parallel_loop(0, n_edges, step=16, unroll=8)(g)

        def fire_scatters(vv, dv, sem, n_edges):
            idx = dv.at[pl.ds(0, n_edges)]
            cps = [pltpu.async_copy(
                vv.at[pl.ds(0, n_edges)], acc_s.at[idx], sem, add=True)]
            if with_deg:
                cps.append(pltpu.async_copy(
                    util.at[pl.ds(0, n_edges)], acc_d.at[idx],
                    sem, add=True))
            return cps

        load_descs = {}
        scat_descs = {}

        def start_loads(ci):
            chunk_id = w * n_pipe + ci
            s, b = ci % NS_, ci % ND_
            d1 = pltpu.async_copy(
                ei_hbm.at[0, pl.ds(chunk_id * _CHUNK, _CHUNK)], srcb[s],
                lsem[s])
            d2 = pltpu.async_copy(
                ei_hbm.at[dyn1, pl.ds(chunk_id * _CHUNK, _CHUNK)], dstb[b],
                dsem[b])
            load_descs[ci] = [d1, d2]

        start_loads(0)
        for ci in range(n_pipe):
            s, b = ci % NS_, ci % ND_
            for d in load_descs.pop(ci):
                d.wait()
            if ci + 1 < n_pipe:
                if ci - 2 >= 0:
                    for d in scat_descs.pop(ci - 2):
                        d.wait()
                start_loads(ci + 1)
            gather_chunk(srcb[s], valb[b], _CHUNK)
            scat_descs[ci] = fire_scatters(valb[b], dstb[b], ssem[b], _CHUNK)
        for ci in sorted(scat_descs):
            for d in scat_descs[ci]:
                d.wait()

        left_id = n_pipe * _NW + w

        def do_tail(n_edges):
            def tail():
                ebase = left_id * _CHUNK
                pltpu.sync_copy(ei_hbm.at[0, pl.ds(ebase, n_edges)],
                                srcb[0].at[pl.ds(0, n_edges)])
                pltpu.sync_copy(ei_hbm.at[dyn1, pl.ds(ebase, n_edges)],
                                dstb[0].at[pl.ds(0, n_edges)])
                gather_chunk(srcb[0], valb[0], n_edges)
                for d in fire_scatters(valb[0], dstb[0], ssem[0], n_edges):
                    d.wait()
            return tail

        if n_left:
            pl.when(left_id < F)(do_tail(_CHUNK))
        if rem:
            pl.when(left_id == F)(do_tail(rem))

        plsc.subcore_barrier()
        pltpu.sync_copy(acc_s.at[pl.ds(base, words)],
                        out_s_hbm.at[cid, pl.ds(base, words)])
        if with_deg:
            pltpu.sync_copy(acc_d.at[pl.ds(base, words)],
                            out_d_hbm.at[cid, pl.ds(base, words)])

    k = pl.kernel(body, out_type=out_type, mesh=_mesh(),
                  scratch_types=scratch,
                  compiler_params=pltpu.CompilerParams(
                      needs_layout_passes=False))
    return k(table, ei)


def _combine(S2, D2, Wl1, Wr1, b1, Wl2, Wr2, b2):
    NP = S2.shape[1]
    H2 = Wl2.shape[1]

    def body(s2, d2, wl1, wr1, b1r, wl2, wr2, b2r,
             mean1_o, gamma_o, dinv_o, u_o):
        s = s2[0, :] + s2[1, :]
        deg = d2[0, :] + d2[1, :]
        dm = jnp.maximum(deg, 1.0)
        mean1_o[:] = s / dm
        gamma_o[:] = deg / dm
        dinv_o[:] = 1.0 / dm
        u1 = jnp.dot(wl1[:], wl2[:])
        u24 = jnp.dot(wr1[:], wl2[:]) + jnp.dot(wl1[:], wr2[:])
        u3 = jnp.dot(b1r[:][None, :], wl2[:])
        u5 = jnp.dot(wr1[:], wr2[:])
        c0 = jnp.dot(b1r[:][None, :], wr2[:]) + b2r[:][None, :]
        z = jnp.zeros((3, H2), jnp.float32)
        u_o[:, :] = jnp.concatenate([u1, u24, u3, u5, c0, z], axis=0)

    return pl.pallas_call(
        body,
        out_shape=[
            jax.ShapeDtypeStruct((NP,), jnp.float32),
            jax.ShapeDtypeStruct((NP,), jnp.float32),
            jax.ShapeDtypeStruct((NP,), jnp.float32),
            jax.ShapeDtypeStruct((8, H2), jnp.float32),
        ],
    )(S2, D2, Wl1, Wr1, b1, Wl2, Wr2, b2)


@functools.partial(jax.jit, static_argnames=("kpad",))
def _pool(A0, A1, mean1, gamma, dinv, xv, cl, U, *, kpad):
    NP = mean1.shape[0]
    per = NP // _NW

    scratch = [
        pltpu.VMEM((per,), jnp.float32),
        pltpu.VMEM((per,), jnp.float32),
        pltpu.VMEM((per,), jnp.float32),
        pltpu.VMEM((per,), jnp.float32),
        pltpu.VMEM((per,), jnp.float32),
        pltpu.VMEM((per,), jnp.float32),
        pltpu.VMEM((per,), jnp.int32),
        pltpu.VMEM((8, 32), jnp.float32),
        pltpu.VMEM((kpad, 32), jnp.float32),
        pltpu.SemaphoreType.DMA,
    ]

    def body(a0_hbm, a1_hbm, m1_hbm, ga_hbm, di_hbm, x_hbm, cl_hbm, u_hbm,
             out_hbm, a0v, a1v, m1v, gav, div, xvv, clv, uv, pooled, sem):
        cid = lax.axis_index("c")
        sid = lax.axis_index("s")
        w = cid * _NS + sid
        base = w * per
        descs = [
            pltpu.async_copy(a0_hbm.at[pl.ds(base, per)], a0v, sem),
            pltpu.async_copy(a1_hbm.at[pl.ds(base, per)], a1v, sem),
            pltpu.async_copy(m1_hbm.at[pl.ds(base, per)], m1v, sem),
            pltpu.async_copy(ga_hbm.at[pl.ds(base, per)], gav, sem),
            pltpu.async_copy(di_hbm.at[pl.ds(base, per)], div, sem),
            pltpu.async_copy(x_hbm.at[pl.ds(base, per)], xvv, sem),
            pltpu.async_copy(cl_hbm.at[pl.ds(base, per)], clv, sem),
            pltpu.async_copy(u_hbm, uv, sem),
        ]

        ninf = jnp.full((16,), -jnp.inf, jnp.float32)

        def pinit(k, _):
            pooled[k, pl.ds(0, 16)] = ninf
            pooled[k, pl.ds(16, 16)] = ninf
            return 0
        lax.fori_loop(0, kpad, pinit, 0)
        for d in descs:
            d.wait()

        u1a = uv[0, pl.ds(0, 16)]
        u1b = uv[0, pl.ds(16, 16)]
        u24a = uv[1, pl.ds(0, 16)]
        u24b = uv[1, pl.ds(16, 16)]
        u3a = uv[2, pl.ds(0, 16)]
        u3b = uv[2, pl.ds(16, 16)]
        u5a = uv[3, pl.ds(0, 16)]
        u5b = uv[3, pl.ds(16, 16)]

        def nb(i, carry):
            cprev, m0, m1 = carry
            b16 = i * 16
            alv = ((a0v[pl.ds(b16, 16)] + a1v[pl.ds(b16, 16)])
                   * div[pl.ds(b16, 16)])
            bev = m1v[pl.ds(b16, 16)]
            gav16 = gav[pl.ds(b16, 16)]
            xxv = xvv[pl.ds(b16, 16)]
            clv16 = clv[pl.ds(b16, 16)]
            for j in range(16):
                al = alv[j]
                be = bev[j]
                ga = gav16[j]
                xx = xxv[j]
                c = clv16[j]
                h0 = (al * u1a + be * u24a) + (ga * u3a + xx * u5a)
                h1 = (al * u1b + be * u24b) + (ga * u3b + xx * u5b)
                flush = c != cprev

                def do_flush(cp=cprev, a=m0, b=m1):
                    pooled[cp, pl.ds(0, 16)] = a
                    pooled[cp, pl.ds(16, 16)] = b
                    return ninf, ninf

                def no_flush(a=m0, b=m1):
                    return a, b
                m0, m1 = lax.cond(flush, do_flush, no_flush)
                m0 = jnp.maximum(m0, h0)
                m1 = jnp.maximum(m1, h1)
                cprev = c
            return cprev, m0, m1
        cprev, m0, m1 = lax.fori_loop(
            0, per // 16, nb,
            (jnp.int32(kpad - 1), ninf, ninf))
        pooled[cprev, pl.ds(0, 16)] = m0
        pooled[cprev, pl.ds(16, 16)] = m1

        pltpu.sync_copy(pooled, out_hbm.at[w])

    k = pl.kernel(
        body,
        out_type=jax.ShapeDtypeStruct((_NW, kpad, 32), jnp.float32),
        mesh=_mesh(),
        scratch_types=scratch,
        compiler_params=pltpu.CompilerParams(needs_layout_passes=False),
    )
    return k(A0, A1, mean1, gamma, dinv, xv, cl, U)


def _head(p32, nclusters, We3, be, Wp1, bp1, Wp2, bp2, U):
    def body(p, we3, ber, wp1, bp1r, wp2, bp2r, u, out):
        pm = jnp.max(p[...], axis=0)
        pm = pm[:nclusters, :]
        pm = jnp.where(jnp.isfinite(pm), pm + u[4, :][None, :], 0.0)
        t = pm[:, :, None] * we3[...]
        emb = jnp.sum(jnp.sum(t, axis=0), axis=0, keepdims=True)
        emb = jnp.maximum(emb + ber[:][None, :], 0.0)
        p1 = jnp.maximum(jnp.dot(emb, wp1[...]) + bp1r[:][None, :], 0.0)
        out[:, :] = jax.nn.sigmoid(jnp.dot(p1, wp2[...]) + bp2r[:][None, :])

    return pl.pallas_call(
        body,
        out_shape=jax.ShapeDtypeStruct((1, 1), jnp.float32),
    )(p32, We3, be, Wp1, bp1, Wp2, bp2, U)


def kernel(x, edge_index, clusters, Wl1, Wr1, b1, Wl2, Wr2, b2,
           We, be, Wp1, bp1, Wp2, bp2):
    N = x.shape[0]
    H2 = Wl2.shape[1]
    K = We.shape[0] // H2
    R = We.shape[1]

    NP = ((N + 2047) // 2048) * 2048
    kpad = ((K + 1 + 15) // 16) * 16

    xv = x[:, 0]
    xp = jnp.concatenate([xv, jnp.zeros((NP - N,), jnp.float32)])
    clp = jnp.concatenate(
        [clusters, jnp.full((NP - N,), K, jnp.int32)])

    S2, D2 = _edge_pass(xv, edge_index, with_deg=True, NP=NP)
    mean1, gamma, dinv, U = _combine(S2, D2, Wl1, Wr1, b1, Wl2, Wr2, b2)
    (A2,) = _edge_pass(mean1, edge_index, with_deg=False, NP=NP)
    p32 = _pool(A2[0], A2[1], mean1, gamma, dinv, xp, clp, U, kpad=kpad)
    We3 = We.reshape(K, H2, R)
    return _head(p32, K, We3, be, Wp1, bp1, Wp2, bp2, U)
